# baseline (speedup 1.0000x reference)
import functools

import jax
import jax.numpy as jnp
from jax import lax
from jax.experimental import pallas as pl
from jax.experimental.pallas import tpu as pltpu
from jax.experimental.pallas import tpu_sc as plsc

SYM_LEN = 100000
MAX_SEQ_LEN = 200
EMB_DIM = 128
BATCH = 1024
SEQ = 200

_HALF = SEQ // 2
_NW = 32
_ROWS_PER_W = BATCH // _NW
_NCHUNK = 2 * _ROWS_PER_W
_NSLOT = 4
_LANES = 16
_VPR = EMB_DIM // _LANES


def _emb_body(x_hbm, sym_hbm, pos_hbm, out_hbm,
              pos_v, idx_v, buf_v, gsem, wsem):
    nc = 2
    wid = lax.axis_index("s") * nc + lax.axis_index("c")
    c0 = wid * _ROWS_PER_W

    pltpu.sync_copy(pos_hbm, pos_v)
    pltpu.sync_copy(x_hbm.at[pl.ds(c0 * 2, _NCHUNK)], idx_v)

    def gather_copy(b, slot):
        return pltpu.make_async_copy(
            sym_hbm.at[idx_v.at[b]],
            buf_v.at[slot],
            gsem.at[slot],
        )

    def wb_copy(b, slot):
        return pltpu.make_async_copy(
            buf_v.at[slot],
            out_hbm.at[c0 + (b >> 1), b & 1],
            wsem.at[slot],
        )

    def add_pos(b, slot):
        h = b & 1

        def row(i, _):
            for j in range(_VPR):
                v = pos_v[h, i, pl.ds(j * _LANES, _LANES)]
                plsc.addupdate(buf_v.at[slot, i, pl.ds(j * _LANES, _LANES)], v)
            return 0

        lax.fori_loop(0, _HALF, row, 0, unroll=False)

    gather_copy(0, 0).start()
    gather_copy(1, 1).start()

    def step(b, _):
        slot = b & (_NSLOT - 1)
        gather_copy(b, slot).wait()
        add_pos(b, slot)
        wb_copy(b, slot).start()

        nb = b + 2
        nslot = nb & (_NSLOT - 1)

        @pl.when(nb < _NCHUNK)
        def _():
            @pl.when(b >= 2)
            def _():
                wb_copy(nb - _NSLOT, nslot).wait()

            gather_copy(nb, nslot).start()

        return 0

    lax.fori_loop(0, _NCHUNK, step, 0, unroll=False)

    for t in range(_NSLOT):
        b = _NCHUNK - _NSLOT + t
        wb_copy(b, b & (_NSLOT - 1)).wait()


@jax.jit
def _emb_call(x3, sym_table, pos3):
    mesh = plsc.VectorSubcoreMesh(core_axis_name="c", subcore_axis_name="s")
    k = functools.partial(
        pl.kernel,
        out_type=jax.ShapeDtypeStruct((BATCH, 2, _HALF, EMB_DIM), jnp.float32),
        mesh=mesh,
        scratch_types=[
            pltpu.VMEM((2, _HALF, EMB_DIM), jnp.float32),
            pltpu.VMEM((_NCHUNK, _HALF), jnp.int32),
            pltpu.VMEM((_NSLOT, _HALF, EMB_DIM), jnp.float32),
            pltpu.SemaphoreType.DMA((_NSLOT,)),
            pltpu.SemaphoreType.DMA((_NSLOT,)),
        ],
    )(_emb_body)
    return k(x3, sym_table, pos3)


def kernel(x, sym_table, pos_table):
    x3 = x.astype(jnp.int32).reshape(2 * BATCH, _HALF)
    pos3 = pos_table.reshape(2, _HALF, EMB_DIM)
    return _emb_call(x3, sym_table, pos3).reshape(BATCH, SEQ, EMB_DIM)


# speedup vs baseline: 1.8213x; 1.8213x over previous
import functools

import jax
import jax.numpy as jnp
from jax import lax
from jax.experimental import pallas as pl
from jax.experimental.pallas import tpu as pltpu
from jax.experimental.pallas import tpu_sc as plsc

SYM_LEN = 100000
MAX_SEQ_LEN = 200
EMB_DIM = 128
BATCH = 1024
SEQ = 200

_HALF = SEQ // 2
_NW = 32
_ROWS_PER_W = BATCH // _NW
_NSLOT = 3
_LANES = 16
_VPR = EMB_DIM // _LANES


def _emb_body(x_hbm, sym_hbm, pos_hbm, out_hbm,
              pos_v, idx_v, buf_v, gsem, wsem):
    nc = 2
    wid = lax.axis_index("s") * nc + lax.axis_index("c")
    c0 = wid * _ROWS_PER_W

    pltpu.sync_copy(pos_hbm, pos_v)
    pltpu.sync_copy(x_hbm.at[pl.ds(c0, _ROWS_PER_W)], idx_v)

    def gather_copies(b, slot):
        return [
            pltpu.make_async_copy(
                sym_hbm.at[idx_v.at[b, h]],
                buf_v.at[slot, pl.ds(h * _HALF, _HALF)],
                gsem.at[slot],
            )
            for h in range(2)
        ]

    def wb_copy(b, slot):
        return pltpu.make_async_copy(
            buf_v.at[slot],
            out_hbm.at[c0 + b],
            wsem.at[slot],
        )

    def add_pos(slot):
        def row(i, _):
            for j in range(_VPR):
                v = pos_v[i, pl.ds(j * _LANES, _LANES)]
                plsc.addupdate(buf_v.at[slot, i, pl.ds(j * _LANES, _LANES)], v)
            return 0

        lax.fori_loop(0, SEQ, row, 0, unroll=False)

    for c in gather_copies(0, 0):
        c.start()

    def step(b, _):
        slot = lax.rem(b, _NSLOT)
        for c in gather_copies(b, slot):
            c.wait()

        nb = b + 1
        nslot = lax.rem(nb, _NSLOT)

        @pl.when(nb < _ROWS_PER_W)
        def _():
            @pl.when(b >= _NSLOT - 1)
            def _():
                wb_copy(nb - _NSLOT, nslot).wait()

            for c in gather_copies(nb, nslot):
                c.start()

        add_pos(slot)
        wb_copy(b, slot).start()
        return 0

    lax.fori_loop(0, _ROWS_PER_W, step, 0, unroll=False)

    for t in range(_NSLOT):
        b = _ROWS_PER_W - _NSLOT + t
        wb_copy(b, lax.rem(jnp.int32(b), _NSLOT)).wait()


@jax.jit
def _emb_call(x3, sym_table, pos_table):
    mesh = plsc.VectorSubcoreMesh(core_axis_name="c", subcore_axis_name="s")
    k = functools.partial(
        pl.kernel,
        out_type=jax.ShapeDtypeStruct((BATCH, SEQ, EMB_DIM), jnp.float32),
        mesh=mesh,
        scratch_types=[
            pltpu.VMEM((MAX_SEQ_LEN, EMB_DIM), jnp.float32),
            pltpu.VMEM((_ROWS_PER_W, 2, _HALF), jnp.int32),
            pltpu.VMEM((_NSLOT, SEQ, EMB_DIM), jnp.float32),
            pltpu.SemaphoreType.DMA((_NSLOT,)),
            pltpu.SemaphoreType.DMA((_NSLOT,)),
        ],
    )(_emb_body)
    return k(x3, sym_table, pos_table)


def kernel(x, sym_table, pos_table):
    x3 = x.astype(jnp.int32).reshape(BATCH, 2, _HALF)
    return _emb_call(x3, sym_table, pos_table)
